# baseline (speedup 1.0000x reference)
import jax
import jax.numpy as jnp
from jax import lax
from jax.experimental import pallas as pl

B, E_IN, C, D, K, V = 256, 768, 8, 64, 1024, 256


def _tc_body(batch_ref, rpT_ref, wT_ref, b_ref, cb_ref, x2_ref, e2_ref,
             val_ref, out_ref):
    c = pl.program_id(0)
    x = batch_ref[...]
    rp = jnp.transpose(rpT_ref[0])
    tq = jnp.dot(x, rp, preferred_element_type=jnp.float32)
    y = lax.dot_general(tq, wT_ref[...], (((1,), (1,)), ((), ())),
                        preferred_element_type=jnp.float32) \
        + b_ref[pl.ds(c, 1), :]
    cb = cb_ref[0]
    xe = lax.dot_general(y, cb, (((1,), (1,)), ((), ())),
                         preferred_element_type=jnp.float32)
    dist = -(x2_ref[0] - 2.0 * xe + e2_ref[0])
    m = jnp.max(dist, axis=1, keepdims=True)
    kidx = lax.broadcasted_iota(jnp.int32, (B, K), 1)
    idx = jnp.min(jnp.where(dist == m, kidx, K), axis=1, keepdims=True)
    vmean = jnp.mean(val_ref[0], axis=1)
    sel = jnp.where(kidx == idx, vmean[None, :], 0.0)
    col = jnp.sum(sel, axis=1, keepdims=True)
    lane = lax.broadcasted_iota(jnp.int32, (B, C), 1)
    out_ref[...] = jnp.where(lane == c, col, out_ref[...])


@jax.jit
def kernel(batch, values, rand_proj, W_in, b_in, codebook):
    tq = jnp.einsum('be,ced->bcd', batch, rand_proj)
    y_full = tq @ W_in.T + b_in
    y4 = y_full.reshape(B, C, C, D)
    x2_full = jnp.sum(y4 * y4, axis=-1)
    x2_diag = jnp.diagonal(x2_full, axis1=1, axis2=2)
    e2 = jnp.sum(codebook * codebook, axis=-1)

    out = pl.pallas_call(
        _tc_body,
        grid=(C,),
        in_specs=[
            pl.BlockSpec((B, E_IN), lambda c: (0, 0)),
            pl.BlockSpec((1, D, E_IN), lambda c: (c, 0, 0)),
            pl.BlockSpec((D, D), lambda c: (c, 0)),
            pl.BlockSpec((C, D), lambda c: (0, 0)),
            pl.BlockSpec((1, K, D), lambda c: (c, 0, 0)),
            pl.BlockSpec((1, B, 1), lambda c: (c, 0, 0)),
            pl.BlockSpec((1, 1, K), lambda c: (c, 0, 0)),
            pl.BlockSpec((1, K, V), lambda c: (c, 0, 0)),
        ],
        out_specs=pl.BlockSpec((B, C), lambda c: (0, 0)),
        out_shape=jax.ShapeDtypeStruct((B, C), jnp.float32),
    )(batch, rand_proj.transpose(0, 2, 1), W_in, b_in.reshape(C, D),
      codebook, x2_diag.T.reshape(C, B, 1), e2.reshape(C, 1, K), values)
    return out


# speedup vs baseline: 1.5365x; 1.5365x over previous
import jax
import jax.numpy as jnp
from jax import lax
from jax.experimental import pallas as pl

B, E_IN, C, D, K, V = 256, 768, 8, 64, 1024, 256


def _tc_body(batch_ref, rpT_ref, wT_ref, b_ref, cbT_ref, val_ref, out_ref):
    c = pl.program_id(0)
    x = batch_ref[...]
    rp = jnp.transpose(rpT_ref[0])
    tq = jnp.dot(x, rp, preferred_element_type=jnp.float32)
    y = lax.dot_general(tq, wT_ref[...], (((1,), (1,)), ((), ())),
                        preferred_element_type=jnp.float32) \
        + b_ref[pl.ds(c, 1), :]
    cbT = cbT_ref[0]
    cb = jnp.transpose(cbT)
    xe = lax.dot_general(y, cb, (((1,), (1,)), ((), ())),
                         preferred_element_type=jnp.float32)

    sq = y * y
    acc8 = sq[:, 0:8]
    for g in range(1, D // 8):
        acc8 = acc8 + sq[:, 8 * g:8 * (g + 1)]
    t4 = acc8[:, 0:4] + acc8[:, 4:8]
    t2 = t4[:, 0:2] + t4[:, 2:4]
    x2 = t2[:, 0:1] + t2[:, 1:2]

    tsq = cbT * cbT
    eacc = tsq[0:8, :]
    for g in range(1, D // 8):
        eacc = eacc + tsq[8 * g:8 * (g + 1), :]
    e4 = eacc[0:4, :] + eacc[4:8, :]
    e2t = e4[0:2, :] + e4[2:4, :]
    e2 = e2t[0:1, :] + e2t[1:2, :]

    dist = -(x2 - 2.0 * xe + e2)
    m = jnp.max(dist, axis=1, keepdims=True)
    kidx = lax.broadcasted_iota(jnp.int32, (B, K), 1)
    idx = jnp.min(jnp.where(dist == m, kidx, K), axis=1, keepdims=True)
    vmean = jnp.mean(val_ref[0], axis=1)
    sel = jnp.where(kidx == idx, vmean[None, :], 0.0)
    col = jnp.sum(sel, axis=1, keepdims=True)
    lane = lax.broadcasted_iota(jnp.int32, (B, C), 1)
    out_ref[...] = jnp.where(lane == c, col, out_ref[...])


@jax.jit
def kernel(batch, values, rand_proj, W_in, b_in, codebook):
    return pl.pallas_call(
        _tc_body,
        grid=(C,),
        in_specs=[
            pl.BlockSpec((B, E_IN), lambda c: (0, 0)),
            pl.BlockSpec((1, D, E_IN), lambda c: (c, 0, 0)),
            pl.BlockSpec((D, D), lambda c: (c, 0)),
            pl.BlockSpec((C, D), lambda c: (0, 0)),
            pl.BlockSpec((1, D, K), lambda c: (c, 0, 0)),
            pl.BlockSpec((1, K, V), lambda c: (c, 0, 0)),
        ],
        out_specs=pl.BlockSpec((B, C), lambda c: (0, 0)),
        out_shape=jax.ShapeDtypeStruct((B, C), jnp.float32),
    )(batch, rand_proj.transpose(0, 2, 1), W_in, b_in.reshape(C, D),
      codebook.transpose(0, 2, 1), values)


# speedup vs baseline: 1.5490x; 1.0082x over previous
import jax
import jax.numpy as jnp
from jax import lax
from jax.experimental import pallas as pl

B, E_IN, C, D, K, V = 256, 768, 8, 64, 1024, 256


def _tc_body(batch_ref, rpT_ref, wT_ref, b_ref, cbT_ref, val_ref, out_ref):
    c = pl.program_id(0)
    x = batch_ref[...]
    tq = lax.dot_general(x, rpT_ref[0], (((1,), (1,)), ((), ())),
                         preferred_element_type=jnp.float32)
    y = lax.dot_general(tq, wT_ref[...], (((1,), (1,)), ((), ())),
                        preferred_element_type=jnp.float32) \
        + b_ref[pl.ds(c, 1), :]
    cbT = cbT_ref[0]
    xe = lax.dot_general(y, cbT, (((1,), (0,)), ((), ())),
                         preferred_element_type=jnp.float32)

    sq = y * y
    acc8 = sq[:, 0:8]
    for g in range(1, D // 8):
        acc8 = acc8 + sq[:, 8 * g:8 * (g + 1)]
    t4 = acc8[:, 0:4] + acc8[:, 4:8]
    t2 = t4[:, 0:2] + t4[:, 2:4]
    x2 = t2[:, 0:1] + t2[:, 1:2]

    tsq = cbT * cbT
    eacc = tsq[0:8, :]
    for g in range(1, D // 8):
        eacc = eacc + tsq[8 * g:8 * (g + 1), :]
    e4 = eacc[0:4, :] + eacc[4:8, :]
    e2t = e4[0:2, :] + e4[2:4, :]
    e2 = e2t[0:1, :] + e2t[1:2, :]

    dist = -(x2 - 2.0 * xe + e2)
    m = jnp.max(dist, axis=1, keepdims=True)
    kidx = lax.broadcasted_iota(jnp.int32, (B, K), 1)
    idx = jnp.min(jnp.where(dist == m, kidx, K), axis=1, keepdims=True)
    vmean = jnp.mean(val_ref[0], axis=1)
    sel = jnp.where(kidx == idx, vmean[None, :], 0.0)
    col = jnp.sum(sel, axis=1, keepdims=True)
    lane = lax.broadcasted_iota(jnp.int32, (B, C), 1)
    out_ref[...] = jnp.where(lane == c, col, out_ref[...])


@jax.jit
def kernel(batch, values, rand_proj, W_in, b_in, codebook):
    return pl.pallas_call(
        _tc_body,
        grid=(C,),
        in_specs=[
            pl.BlockSpec((B, E_IN), lambda c: (0, 0)),
            pl.BlockSpec((1, D, E_IN), lambda c: (c, 0, 0)),
            pl.BlockSpec((D, D), lambda c: (c, 0)),
            pl.BlockSpec((C, D), lambda c: (0, 0)),
            pl.BlockSpec((1, D, K), lambda c: (c, 0, 0)),
            pl.BlockSpec((1, K, V), lambda c: (c, 0, 0)),
        ],
        out_specs=pl.BlockSpec((B, C), lambda c: (0, 0)),
        out_shape=jax.ShapeDtypeStruct((B, C), jnp.float32),
    )(batch, rand_proj.transpose(0, 2, 1), W_in, b_in.reshape(C, D),
      codebook.transpose(0, 2, 1), values)


# speedup vs baseline: 1.6368x; 1.0566x over previous
import jax
import jax.numpy as jnp
from jax import lax
from jax.experimental import pallas as pl

B, E_IN, C, D, K, V = 256, 768, 8, 64, 1024, 256


def _tc_body(batch_ref, rpT_ref, wT_ref, b_ref, cbT_ref, val_ref, out_ref):
    c = pl.program_id(0)
    x = batch_ref[...]
    tq = lax.dot_general(x, rpT_ref[0], (((1,), (1,)), ((), ())),
                         preferred_element_type=jnp.float32)
    b128 = b_ref[pl.ds((c // 2) * 2 * D, 2 * D)]
    bseg = jnp.where(c % 2 == 0, b128[:D], b128[D:])
    y = lax.dot_general(tq, wT_ref[...], (((1,), (1,)), ((), ())),
                        preferred_element_type=jnp.float32) + bseg[None, :]
    cbT = cbT_ref[0]
    xe = lax.dot_general(y, cbT, (((1,), (0,)), ((), ())),
                         preferred_element_type=jnp.float32)

    sq = y * y
    acc8 = sq[:, 0:8]
    for g in range(1, D // 8):
        acc8 = acc8 + sq[:, 8 * g:8 * (g + 1)]
    t4 = acc8[:, 0:4] + acc8[:, 4:8]
    t2 = t4[:, 0:2] + t4[:, 2:4]
    x2 = t2[:, 0:1] + t2[:, 1:2]

    tsq = cbT * cbT
    eacc = tsq[0:8, :]
    for g in range(1, D // 8):
        eacc = eacc + tsq[8 * g:8 * (g + 1), :]
    e4 = eacc[0:4, :] + eacc[4:8, :]
    e2t = e4[0:2, :] + e4[2:4, :]
    e2 = e2t[0:1, :] + e2t[1:2, :]

    dist = -(x2 - 2.0 * xe + e2)
    m = jnp.max(dist, axis=1, keepdims=True)
    kidx = lax.broadcasted_iota(jnp.int32, (B, K), 1)
    idx = jnp.min(jnp.where(dist == m, kidx, K), axis=1, keepdims=True)
    vmean = jnp.mean(val_ref[0], axis=1)
    sel = jnp.where(kidx == idx, vmean[None, :], 0.0)
    col = jnp.sum(sel, axis=1, keepdims=True)
    lane = lax.broadcasted_iota(jnp.int32, (B, C), 1)
    out_ref[...] = jnp.where(lane == c, col, out_ref[...])


@jax.jit
def kernel(batch, values, rand_proj, W_in, b_in, codebook):
    return pl.pallas_call(
        _tc_body,
        grid=(C,),
        in_specs=[
            pl.BlockSpec((B, E_IN), lambda c: (0, 0)),
            pl.BlockSpec((1, D, E_IN), lambda c: (c, 0, 0)),
            pl.BlockSpec((D, D), lambda c: (c, 0)),
            pl.BlockSpec((C * D,), lambda c: (0,)),
            pl.BlockSpec((1, D, K), lambda c: (c, 0, 0)),
            pl.BlockSpec((1, K, V), lambda c: (c, 0, 0)),
        ],
        out_specs=pl.BlockSpec((B, C), lambda c: (0, 0)),
        out_shape=jax.ShapeDtypeStruct((B, C), jnp.float32),
    )(batch, rand_proj.transpose(0, 2, 1), W_in, b_in,
      codebook.transpose(0, 2, 1), values)
